# baseline (speedup 1.0000x reference)
import functools

import jax
import jax.numpy as jnp
from jax import lax
from jax.experimental import pallas as pl
from jax.experimental.pallas import tpu as pltpu
from jax.experimental.pallas import tpu_sc as plsc

_N = 10000
_E = 320000
_D = 128
_NC, _NS, _L = 2, 16, 16
_NW = _NC * _NS
_RPT = 640
_NP = _NS * _RPT
_BLK = 64
_NBLK = 160
_EP = _BLK * _NBLK * _NW
_DUMMY = _N


def _sc_mesh():
    return plsc.VectorSubcoreMesh(
        core_axis_name="c", subcore_axis_name="s",
        num_cores=_NC, num_subcores=_NS)


@functools.cache
def _sc_degree_kernel():
    return functools.partial(
        pl.kernel,
        out_type=jax.ShapeDtypeStruct((_NC, _NP), jnp.float32),
        mesh=_sc_mesh(),
        scratch_types=[
            pltpu.VMEM((_NBLK, _BLK), jnp.int32),
            pltpu.VMEM((_RPT,), jnp.float32),
            pltpu.VMEM((_BLK,), jnp.float32),
            pltpu.VMEM_SHARED((_NP,), jnp.float32),
        ],
    )(_sc_degree_body)


def _sc_degree(dstp):
    return _sc_degree_kernel()(dstp)


def _sc_degree_body(dst_hbm, out_hbm, dst_v, z_v, one_v, acc_sh):
    c = lax.axis_index("c")
    s = lax.axis_index("s")
    wid = s * _NC + c
    pltpu.sync_copy(dst_hbm.at[wid], dst_v)

    def _zero(t, carry):
        z_v[pl.ds(t * _L, _L)] = jnp.zeros((_L,), jnp.float32)
        return carry
    lax.fori_loop(0, _RPT // _L, _zero, None, unroll=8)

    def _ones(t, carry):
        one_v[pl.ds(t * _L, _L)] = jnp.ones((_L,), jnp.float32)
        return carry
    lax.fori_loop(0, _BLK // _L, _ones, None, unroll=8)

    pltpu.sync_copy(z_v, acc_sh.at[pl.ds(s * _RPT, _RPT)])
    plsc.subcore_barrier()

    def _count(b, carry):
        pltpu.sync_copy(one_v, acc_sh.at[dst_v.at[b]], add=True)
        return carry
    lax.fori_loop(0, _NBLK, _count, None)

    plsc.subcore_barrier()
    pltpu.sync_copy(acc_sh.at[pl.ds(s * _RPT, _RPT)],
                    out_hbm.at[c, pl.ds(s * _RPT, _RPT)])


_GB = 16
_NBUF = 4
_RW = _D // 2
_NBLK_PAIR = 2 * _NBLK
_NBLK_C0 = 240
_NBLK_C1 = _NBLK_PAIR - _NBLK_C0


@functools.cache
def _sc_scatter_kernel():
    return functools.partial(
        pl.kernel,
        out_type=jax.ShapeDtypeStruct((_NC, _NP, _D), jnp.float32),
        mesh=_sc_mesh(),
        scratch_types=[
            pltpu.VMEM((2, _GB, _BLK), jnp.int32),
            pltpu.VMEM((2, _GB, _BLK), jnp.int32),
            pltpu.VMEM((_NBUF, _BLK, _RW), jnp.int32),
            pltpu.VMEM((2, _BLK, _D), jnp.float32),
            pltpu.VMEM_SHARED((_NP, _D), jnp.float32),
        ] + [pltpu.SemaphoreType.DMA] * (_NBUF + 3),
        compiler_params=pltpu.CompilerParams(use_tc_tiling_on_sc=False,
                                             needs_layout_passes=False),
    )(_sc_scatter_body)


def _sc_scatter_rows(gp, srcp, dstp):
    return _sc_scatter_kernel()(gp, srcp, dstp)


def _sc_scatter_body(g_hbm, src_hbm, dst_hbm, out_hbm,
                     src_v, dst_v, rows_v, frows_v, acc_sh, *sems):
    gsems, ssems, sem_i = sems[:_NBUF], sems[_NBUF:_NBUF + 2], sems[_NBUF + 2]
    c = lax.axis_index("c")
    s = lax.axis_index("s")
    blk0 = s * _NBLK_PAIR + lax.select(c == 0, 0, _NBLK_C0)
    ng = lax.select(c == 0, _NBLK_C0 // _GB, _NBLK_C1 // _GB)

    def _zero(t, carry):
        frows_v[0, t // (_D // _L), pl.ds((t % (_D // _L)) * _L, _L)] = (
            jnp.zeros((_L,), jnp.float32))
        return carry
    lax.fori_loop(0, _BLK * (_D // _L), _zero, None, unroll=8)
    for k in range(_RPT // _BLK):
        pltpu.sync_copy(frows_v.at[0],
                        acc_sh.at[pl.ds(s * _RPT + k * _BLK, _BLK)])
    plsc.subcore_barrier()

    pltpu.sync_copy(src_hbm.at[pl.ds(blk0, _GB)], src_v.at[0])
    pltpu.sync_copy(dst_hbm.at[pl.ds(blk0, _GB)], dst_v.at[0])

    def _group(gidx, carry):
        buf = gidx % 2
        nxt = 1 - buf

        @pl.when(gidx < ng - 1)
        def _():
            pltpu.async_copy(
                src_hbm.at[pl.ds(blk0 + (gidx + 1) * _GB, _GB)],
                src_v.at[nxt], sem_i)
            pltpu.async_copy(
                dst_hbm.at[pl.ds(blk0 + (gidx + 1) * _GB, _GB)],
                dst_v.at[nxt], sem_i)

        for k in range(_NBUF - 1):
            pltpu.async_copy(g_hbm.at[src_v.at[buf, k]], rows_v.at[k],
                             gsems[k])
        for b in range(_GB):
            k = b % _NBUF
            fb = b % 2
            if b + _NBUF - 1 < _GB:
                kr = (b + _NBUF - 1) % _NBUF
                pltpu.async_copy(g_hbm.at[src_v.at[buf, b + _NBUF - 1]],
                                 rows_v.at[kr], gsems[kr])
            pltpu.make_async_copy(g_hbm.at[src_v.at[buf, b]], rows_v.at[k],
                                  gsems[k]).wait()
            if b >= 2:
                pltpu.make_async_copy(frows_v.at[fb],
                                      acc_sh.at[dst_v.at[buf, b]],
                                      ssems[fb]).wait()

            def _conv(r, carry2, _k=k, _fb=fb):
                for j in range(_RW // _L):
                    v = rows_v[_k, r, pl.ds(j * _L, _L)]
                    frows_v[_fb, r, pl.ds(j * _L, _L)] = plsc.bitcast(
                        v << 16, jnp.float32)
                    frows_v[_fb, r, pl.ds(_RW + j * _L, _L)] = plsc.bitcast(
                        v & jnp.int32(-65536), jnp.float32)
                return carry2
            lax.fori_loop(0, _BLK, _conv, None, unroll=4)

            pltpu.async_copy(frows_v.at[fb], acc_sh.at[dst_v.at[buf, b]],
                             ssems[fb], add=True)
        for fb in range(2):
            pltpu.make_async_copy(frows_v.at[fb],
                                  acc_sh.at[dst_v.at[buf, 0]],
                                  ssems[fb]).wait()

        @pl.when(gidx < ng - 1)
        def _():
            pltpu.make_async_copy(src_hbm.at[pl.ds(0, _GB)],
                                  src_v.at[nxt], sem_i).wait()
            pltpu.make_async_copy(dst_hbm.at[pl.ds(0, _GB)],
                                  dst_v.at[nxt], sem_i).wait()
        return carry
    lax.fori_loop(0, ng, _group, None)

    plsc.subcore_barrier()
    pltpu.sync_copy(acc_sh.at[pl.ds(s * _RPT, _RPT)],
                    out_hbm.at[c, pl.ds(s * _RPT, _RPT)])


_R = 1024


def _pack_bf16(g):
    lo = lax.bitcast_convert_type(
        g[:, :_RW].astype(jnp.bfloat16), jnp.uint16).astype(jnp.uint32)
    hi = lax.bitcast_convert_type(
        g[:, _RW:].astype(jnp.bfloat16), jnp.uint16).astype(jnp.uint32)
    return lax.bitcast_convert_type(lo | (hi << 16), jnp.int32)


def _tc_first(cntT, xp, W1):
    def body(cnt_ref, x_ref, w_ref, dinv_ref, g_ref, gp_ref):
        deg = 1.0 + cnt_ref[:, 0:1] + cnt_ref[:, 1:2]
        dinv = lax.rsqrt(deg)
        dinv_ref[...] = dinv
        g = jnp.dot(x_ref[...], w_ref[...],
                    preferred_element_type=jnp.float32) * dinv
        g_ref[...] = g
        gp_ref[...] = _pack_bf16(g)
    return pl.pallas_call(
        body,
        grid=(_NP // _R,),
        in_specs=[pl.BlockSpec((_R, 2), lambda i: (i, 0)),
                  pl.BlockSpec((_R, _D), lambda i: (i, 0)),
                  pl.BlockSpec((_D, _D), lambda i: (0, 0))],
        out_specs=[pl.BlockSpec((_R, 1), lambda i: (i, 0)),
                   pl.BlockSpec((_R, _D), lambda i: (i, 0)),
                   pl.BlockSpec((_R, _RW), lambda i: (i, 0))],
        out_shape=[jax.ShapeDtypeStruct((_NP, 1), jnp.float32),
                   jax.ShapeDtypeStruct((_NP, _D), jnp.float32),
                   jax.ShapeDtypeStruct((_NP, _RW), jnp.int32)],
    )(cntT, xp, W1)


def _leaky(v):
    return jnp.where(v >= 0, v, 0.2 * v)


def _tc_combine(sp, g, dinv, b, hprev, Wn):
    has_res = hprev is not None

    def body(*refs):
        if has_res:
            (s_ref, g_ref, dinv_ref, b_ref, h_ref, w_ref,
             hout_ref, gout_ref, gpout_ref) = refs
        else:
            (s_ref, g_ref, dinv_ref, b_ref, w_ref,
             hout_ref, gout_ref, gpout_ref) = refs
        dinv = dinv_ref[...]
        conv = dinv * (s_ref[0] + s_ref[1] + g_ref[...]) + b_ref[...]
        pre = h_ref[...] + conv if has_res else conv
        h = _leaky(pre)
        hout_ref[...] = h
        gout = jnp.dot(h, w_ref[...],
                       preferred_element_type=jnp.float32) * dinv
        gout_ref[...] = gout
        gpout_ref[...] = _pack_bf16(gout)

    in_specs = [pl.BlockSpec((_NC, _R, _D), lambda i: (0, i, 0)),
                pl.BlockSpec((_R, _D), lambda i: (i, 0)),
                pl.BlockSpec((_R, 1), lambda i: (i, 0)),
                pl.BlockSpec((1, _D), lambda i: (0, 0))]
    args = [sp, g, dinv, b]
    if has_res:
        in_specs.append(pl.BlockSpec((_R, _D), lambda i: (i, 0)))
        args.append(hprev)
    in_specs.append(pl.BlockSpec((_D, _D), lambda i: (0, 0)))
    args.append(Wn)
    return pl.pallas_call(
        body,
        grid=(_NP // _R,),
        in_specs=in_specs,
        out_specs=[pl.BlockSpec((_R, _D), lambda i: (i, 0)),
                   pl.BlockSpec((_R, _D), lambda i: (i, 0)),
                   pl.BlockSpec((_R, _RW), lambda i: (i, 0))],
        out_shape=[jax.ShapeDtypeStruct((_NP, _D), jnp.float32),
                   jax.ShapeDtypeStruct((_NP, _D), jnp.float32),
                   jax.ShapeDtypeStruct((_NP, _RW), jnp.int32)],
    )(*args)


def _tc_final(sp, g, dinv, b, hprev, Wl, bl):
    def body(s_ref, g_ref, dinv_ref, b_ref, h_ref, wl_ref, bl_ref, out_ref):
        dinv = dinv_ref[...]
        conv = dinv * (s_ref[0] + s_ref[1] + g_ref[...]) + b_ref[...]
        h = _leaky(h_ref[...] + conv)
        out_ref[...] = jnp.dot(h, wl_ref[...],
                               preferred_element_type=jnp.float32) + bl_ref[...]
    return pl.pallas_call(
        body,
        grid=(_NP // _R,),
        in_specs=[pl.BlockSpec((_NC, _R, _D), lambda i: (0, i, 0)),
                  pl.BlockSpec((_R, _D), lambda i: (i, 0)),
                  pl.BlockSpec((_R, 1), lambda i: (i, 0)),
                  pl.BlockSpec((1, _D), lambda i: (0, 0)),
                  pl.BlockSpec((_R, _D), lambda i: (i, 0)),
                  pl.BlockSpec((_D, 1), lambda i: (0, 0)),
                  pl.BlockSpec((1, 1), lambda i: (0, 0))],
        out_specs=pl.BlockSpec((_R, 1), lambda i: (i, 0)),
        out_shape=jax.ShapeDtypeStruct((_NP, 1), jnp.float32),
    )(sp, g, dinv, b, hprev, Wl, bl)


def kernel(x, edge_index, W1, b1, W2, b2, W3, b3, Wl, bl):
    pad = jnp.full((_EP - _E,), _DUMMY, dtype=jnp.int32)
    src_flat = jnp.concatenate([edge_index[0], pad])
    dst_flat = jnp.concatenate([edge_index[1], pad])
    srcp = src_flat.reshape(_NS * _NBLK_PAIR, _BLK)
    dstp = dst_flat.reshape(_NS * _NBLK_PAIR, _BLK)
    xp = jnp.pad(x, ((0, _NP - _N), (0, 0)))

    cnt = _sc_degree(dst_flat.reshape(_NW, _NBLK, _BLK))
    dinv, g1, gp1 = _tc_first(cnt.T, xp, W1)
    s1 = _sc_scatter_rows(gp1, srcp, dstp)
    h1, g2, gp2 = _tc_combine(s1, g1, dinv, b1.reshape(1, _D), None, W2)
    s2 = _sc_scatter_rows(gp2, srcp, dstp)
    h2, g3, gp3 = _tc_combine(s2, g2, dinv, b2.reshape(1, _D), h1, W3)
    s3 = _sc_scatter_rows(gp3, srcp, dstp)
    logits = _tc_final(s3, g3, dinv, b3.reshape(1, _D), h2, Wl,
                       bl.reshape(1, 1))
    return logits[:_N, 0]


# speedup vs baseline: 1.0992x; 1.0992x over previous
import functools

import jax
import jax.numpy as jnp
from jax import lax
from jax.experimental import pallas as pl
from jax.experimental.pallas import tpu as pltpu
from jax.experimental.pallas import tpu_sc as plsc

_N = 10000
_E = 320000
_D = 128
_NC, _NS, _L = 2, 16, 16
_NW = _NC * _NS
_RPT = 640
_NP = _NS * _RPT
_BLK = 64
_NBLK = 160
_EP = _BLK * _NBLK * _NW
_DUMMY = _N


def _sc_mesh():
    return plsc.VectorSubcoreMesh(
        core_axis_name="c", subcore_axis_name="s",
        num_cores=_NC, num_subcores=_NS)


@functools.cache
def _sc_degree_kernel():
    return functools.partial(
        pl.kernel,
        out_type=jax.ShapeDtypeStruct((_NC, _NP), jnp.float32),
        mesh=_sc_mesh(),
        scratch_types=[
            pltpu.VMEM((_NBLK, _BLK), jnp.int32),
            pltpu.VMEM((_RPT,), jnp.float32),
            pltpu.VMEM((_BLK,), jnp.float32),
            pltpu.VMEM_SHARED((_NP,), jnp.float32),
        ],
    )(_sc_degree_body)


def _sc_degree(dstp):
    return _sc_degree_kernel()(dstp)


def _sc_degree_body(dst_hbm, out_hbm, dst_v, z_v, one_v, acc_sh):
    c = lax.axis_index("c")
    s = lax.axis_index("s")
    wid = s * _NC + c
    pltpu.sync_copy(dst_hbm.at[wid], dst_v)

    def _zero(t, carry):
        z_v[pl.ds(t * _L, _L)] = jnp.zeros((_L,), jnp.float32)
        return carry
    lax.fori_loop(0, _RPT // _L, _zero, None, unroll=8)

    def _ones(t, carry):
        one_v[pl.ds(t * _L, _L)] = jnp.ones((_L,), jnp.float32)
        return carry
    lax.fori_loop(0, _BLK // _L, _ones, None, unroll=8)

    pltpu.sync_copy(z_v, acc_sh.at[pl.ds(s * _RPT, _RPT)])
    plsc.subcore_barrier()

    def _count(b, carry):
        pltpu.sync_copy(one_v, acc_sh.at[dst_v.at[b]], add=True)
        return carry
    lax.fori_loop(0, _NBLK, _count, None)

    plsc.subcore_barrier()
    pltpu.sync_copy(acc_sh.at[pl.ds(s * _RPT, _RPT)],
                    out_hbm.at[c, pl.ds(s * _RPT, _RPT)])


_GB = 16
_NBUF = 4
_RW = _D // 2
_NBLK_PAIR = 2 * _NBLK
_NBLK_C0 = 160
_NBLK_C1 = _NBLK_PAIR - _NBLK_C0


@functools.cache
def _sc_scatter_kernel():
    return functools.partial(
        pl.kernel,
        out_type=jax.ShapeDtypeStruct((_NC, _NP, _D), jnp.float32),
        mesh=_sc_mesh(),
        scratch_types=[
            pltpu.VMEM((2, _GB, _BLK), jnp.int32),
            pltpu.VMEM((2, _GB, _BLK), jnp.int32),
            pltpu.VMEM((_NBUF, _BLK, _RW), jnp.int32),
            pltpu.VMEM((2, _BLK, _D), jnp.float32),
            pltpu.VMEM_SHARED((_NP, _D), jnp.float32),
        ] + [pltpu.SemaphoreType.DMA] * (_NBUF + 3),
        compiler_params=pltpu.CompilerParams(use_tc_tiling_on_sc=False,
                                             needs_layout_passes=False),
    )(_sc_scatter_body)


def _sc_scatter_rows(gp, srcp, dstp):
    return _sc_scatter_kernel()(gp, srcp, dstp)


def _sc_scatter_body(g_hbm, src_hbm, dst_hbm, out_hbm,
                     src_v, dst_v, rows_v, frows_v, acc_sh, *sems):
    gsems, ssems, sem_i = sems[:_NBUF], sems[_NBUF:_NBUF + 2], sems[_NBUF + 2]
    c = lax.axis_index("c")
    s = lax.axis_index("s")
    blk0 = s * _NBLK_PAIR + lax.select(c == 0, 0, _NBLK_C0)
    ng = lax.select(c == 0, _NBLK_C0 // _GB, _NBLK_C1 // _GB)

    def _zero(t, carry):
        frows_v[0, t // (_D // _L), pl.ds((t % (_D // _L)) * _L, _L)] = (
            jnp.zeros((_L,), jnp.float32))
        return carry
    lax.fori_loop(0, _BLK * (_D // _L), _zero, None, unroll=8)
    for k in range(_RPT // _BLK):
        pltpu.sync_copy(frows_v.at[0],
                        acc_sh.at[pl.ds(s * _RPT + k * _BLK, _BLK)])
    plsc.subcore_barrier()

    pltpu.sync_copy(src_hbm.at[pl.ds(blk0, _GB)], src_v.at[0])
    pltpu.sync_copy(dst_hbm.at[pl.ds(blk0, _GB)], dst_v.at[0])

    def _group(gidx, carry):
        buf = gidx % 2
        nxt = 1 - buf

        @pl.when(gidx < ng - 1)
        def _():
            pltpu.async_copy(
                src_hbm.at[pl.ds(blk0 + (gidx + 1) * _GB, _GB)],
                src_v.at[nxt], sem_i)
            pltpu.async_copy(
                dst_hbm.at[pl.ds(blk0 + (gidx + 1) * _GB, _GB)],
                dst_v.at[nxt], sem_i)

        for k in range(_NBUF - 1):
            pltpu.async_copy(g_hbm.at[src_v.at[buf, k]], rows_v.at[k],
                             gsems[k])
        for b in range(_GB):
            k = b % _NBUF
            fb = b % 2
            if b + _NBUF - 1 < _GB:
                kr = (b + _NBUF - 1) % _NBUF
                pltpu.async_copy(g_hbm.at[src_v.at[buf, b + _NBUF - 1]],
                                 rows_v.at[kr], gsems[kr])
            pltpu.make_async_copy(g_hbm.at[src_v.at[buf, b]], rows_v.at[k],
                                  gsems[k]).wait()
            if b >= 2:
                pltpu.make_async_copy(frows_v.at[fb],
                                      acc_sh.at[dst_v.at[buf, b]],
                                      ssems[fb]).wait()

            def _conv(r, carry2, _k=k, _fb=fb):
                for j in range(_RW // _L):
                    v = rows_v[_k, r, pl.ds(j * _L, _L)]
                    frows_v[_fb, r, pl.ds(j * _L, _L)] = plsc.bitcast(
                        v << 16, jnp.float32)
                    frows_v[_fb, r, pl.ds(_RW + j * _L, _L)] = plsc.bitcast(
                        v & jnp.int32(-65536), jnp.float32)
                return carry2
            lax.fori_loop(0, _BLK, _conv, None, unroll=4)

            pltpu.async_copy(frows_v.at[fb], acc_sh.at[dst_v.at[buf, b]],
                             ssems[fb], add=True)
        for fb in range(2):
            pltpu.make_async_copy(frows_v.at[fb],
                                  acc_sh.at[dst_v.at[buf, 0]],
                                  ssems[fb]).wait()

        @pl.when(gidx < ng - 1)
        def _():
            pltpu.make_async_copy(src_hbm.at[pl.ds(0, _GB)],
                                  src_v.at[nxt], sem_i).wait()
            pltpu.make_async_copy(dst_hbm.at[pl.ds(0, _GB)],
                                  dst_v.at[nxt], sem_i).wait()
        return carry
    lax.fori_loop(0, ng, _group, None)

    plsc.subcore_barrier()
    pltpu.sync_copy(acc_sh.at[pl.ds(s * _RPT, _RPT)],
                    out_hbm.at[c, pl.ds(s * _RPT, _RPT)])


_R = 1024


def _pack_bf16(g):
    lo = lax.bitcast_convert_type(
        g[:, :_RW].astype(jnp.bfloat16), jnp.uint16).astype(jnp.uint32)
    hi = lax.bitcast_convert_type(
        g[:, _RW:].astype(jnp.bfloat16), jnp.uint16).astype(jnp.uint32)
    return lax.bitcast_convert_type(lo | (hi << 16), jnp.int32)


def _tc_first(cntT, xp, W1):
    def body(cnt_ref, x_ref, w_ref, dinv_ref, g_ref, gp_ref):
        deg = 1.0 + cnt_ref[:, 0:1] + cnt_ref[:, 1:2]
        dinv = lax.rsqrt(deg)
        dinv_ref[...] = dinv
        g = jnp.dot(x_ref[...], w_ref[...],
                    preferred_element_type=jnp.float32) * dinv
        g_ref[...] = g
        gp_ref[...] = _pack_bf16(g)
    return pl.pallas_call(
        body,
        grid=(_NP // _R,),
        in_specs=[pl.BlockSpec((_R, 2), lambda i: (i, 0)),
                  pl.BlockSpec((_R, _D), lambda i: (i, 0)),
                  pl.BlockSpec((_D, _D), lambda i: (0, 0))],
        out_specs=[pl.BlockSpec((_R, 1), lambda i: (i, 0)),
                   pl.BlockSpec((_R, _D), lambda i: (i, 0)),
                   pl.BlockSpec((_R, _RW), lambda i: (i, 0))],
        out_shape=[jax.ShapeDtypeStruct((_NP, 1), jnp.float32),
                   jax.ShapeDtypeStruct((_NP, _D), jnp.float32),
                   jax.ShapeDtypeStruct((_NP, _RW), jnp.int32)],
    )(cntT, xp, W1)


def _leaky(v):
    return jnp.where(v >= 0, v, 0.2 * v)


def _tc_combine(sp, g, dinv, b, hprev, Wn):
    has_res = hprev is not None

    def body(*refs):
        if has_res:
            (s_ref, g_ref, dinv_ref, b_ref, h_ref, w_ref,
             hout_ref, gout_ref, gpout_ref) = refs
        else:
            (s_ref, g_ref, dinv_ref, b_ref, w_ref,
             hout_ref, gout_ref, gpout_ref) = refs
        dinv = dinv_ref[...]
        conv = dinv * (s_ref[0] + s_ref[1] + g_ref[...]) + b_ref[...]
        pre = h_ref[...] + conv if has_res else conv
        h = _leaky(pre)
        hout_ref[...] = h
        gout = jnp.dot(h, w_ref[...],
                       preferred_element_type=jnp.float32) * dinv
        gout_ref[...] = gout
        gpout_ref[...] = _pack_bf16(gout)

    in_specs = [pl.BlockSpec((_NC, _R, _D), lambda i: (0, i, 0)),
                pl.BlockSpec((_R, _D), lambda i: (i, 0)),
                pl.BlockSpec((_R, 1), lambda i: (i, 0)),
                pl.BlockSpec((1, _D), lambda i: (0, 0))]
    args = [sp, g, dinv, b]
    if has_res:
        in_specs.append(pl.BlockSpec((_R, _D), lambda i: (i, 0)))
        args.append(hprev)
    in_specs.append(pl.BlockSpec((_D, _D), lambda i: (0, 0)))
    args.append(Wn)
    return pl.pallas_call(
        body,
        grid=(_NP // _R,),
        in_specs=in_specs,
        out_specs=[pl.BlockSpec((_R, _D), lambda i: (i, 0)),
                   pl.BlockSpec((_R, _D), lambda i: (i, 0)),
                   pl.BlockSpec((_R, _RW), lambda i: (i, 0))],
        out_shape=[jax.ShapeDtypeStruct((_NP, _D), jnp.float32),
                   jax.ShapeDtypeStruct((_NP, _D), jnp.float32),
                   jax.ShapeDtypeStruct((_NP, _RW), jnp.int32)],
    )(*args)


def _tc_final(sp, g, dinv, b, hprev, Wl, bl):
    def body(s_ref, g_ref, dinv_ref, b_ref, h_ref, wl_ref, bl_ref, out_ref):
        dinv = dinv_ref[...]
        conv = dinv * (s_ref[0] + s_ref[1] + g_ref[...]) + b_ref[...]
        h = _leaky(h_ref[...] + conv)
        out_ref[...] = jnp.dot(h, wl_ref[...],
                               preferred_element_type=jnp.float32) + bl_ref[...]
    return pl.pallas_call(
        body,
        grid=(_NP // _R,),
        in_specs=[pl.BlockSpec((_NC, _R, _D), lambda i: (0, i, 0)),
                  pl.BlockSpec((_R, _D), lambda i: (i, 0)),
                  pl.BlockSpec((_R, 1), lambda i: (i, 0)),
                  pl.BlockSpec((1, _D), lambda i: (0, 0)),
                  pl.BlockSpec((_R, _D), lambda i: (i, 0)),
                  pl.BlockSpec((_D, 1), lambda i: (0, 0)),
                  pl.BlockSpec((1, 1), lambda i: (0, 0))],
        out_specs=pl.BlockSpec((_R, 1), lambda i: (i, 0)),
        out_shape=jax.ShapeDtypeStruct((_NP, 1), jnp.float32),
    )(sp, g, dinv, b, hprev, Wl, bl)


def kernel(x, edge_index, W1, b1, W2, b2, W3, b3, Wl, bl):
    pad = jnp.full((_EP - _E,), _DUMMY, dtype=jnp.int32)
    src_flat = jnp.concatenate([edge_index[0], pad])
    dst_flat = jnp.concatenate([edge_index[1], pad])
    srcp = src_flat.reshape(_NS * _NBLK_PAIR, _BLK)
    dstp = dst_flat.reshape(_NS * _NBLK_PAIR, _BLK)
    xp = jnp.pad(x, ((0, _NP - _N), (0, 0)))

    cnt = _sc_degree(dst_flat.reshape(_NW, _NBLK, _BLK))
    dinv, g1, gp1 = _tc_first(cnt.T, xp, W1)
    s1 = _sc_scatter_rows(gp1, srcp, dstp)
    h1, g2, gp2 = _tc_combine(s1, g1, dinv, b1.reshape(1, _D), None, W2)
    s2 = _sc_scatter_rows(gp2, srcp, dstp)
    h2, g3, gp3 = _tc_combine(s2, g2, dinv, b2.reshape(1, _D), h1, W3)
    s3 = _sc_scatter_rows(gp3, srcp, dstp)
    logits = _tc_final(s3, g3, dinv, b3.reshape(1, _D), h2, Wl,
                       bl.reshape(1, 1))
    return logits[:_N, 0]


# speedup vs baseline: 1.1015x; 1.0021x over previous
import functools

import jax
import jax.numpy as jnp
from jax import lax
from jax.experimental import pallas as pl
from jax.experimental.pallas import tpu as pltpu
from jax.experimental.pallas import tpu_sc as plsc

_N = 10000
_E = 320000
_D = 128
_NC, _NS, _L = 2, 16, 16
_NW = _NC * _NS
_RPT = 640
_NP = _NS * _RPT
_BLK = 64
_NBLK = 160
_EP = _BLK * _NBLK * _NW
_DUMMY = _N


def _sc_mesh():
    return plsc.VectorSubcoreMesh(
        core_axis_name="c", subcore_axis_name="s",
        num_cores=_NC, num_subcores=_NS)


@functools.cache
def _sc_degree_kernel():
    return functools.partial(
        pl.kernel,
        out_type=jax.ShapeDtypeStruct((_NC, _NP), jnp.float32),
        mesh=_sc_mesh(),
        scratch_types=[
            pltpu.VMEM((_NBLK, _BLK), jnp.int32),
            pltpu.VMEM((_RPT,), jnp.float32),
            pltpu.VMEM((_BLK,), jnp.float32),
            pltpu.VMEM_SHARED((_NP,), jnp.float32),
        ],
    )(_sc_degree_body)


def _sc_degree(dstp):
    return _sc_degree_kernel()(dstp)


def _sc_degree_body(dst_hbm, out_hbm, dst_v, z_v, one_v, acc_sh):
    c = lax.axis_index("c")
    s = lax.axis_index("s")
    wid = s * _NC + c
    pltpu.sync_copy(dst_hbm.at[wid], dst_v)

    def _zero(t, carry):
        z_v[pl.ds(t * _L, _L)] = jnp.zeros((_L,), jnp.float32)
        return carry
    lax.fori_loop(0, _RPT // _L, _zero, None, unroll=8)

    def _ones(t, carry):
        one_v[pl.ds(t * _L, _L)] = jnp.ones((_L,), jnp.float32)
        return carry
    lax.fori_loop(0, _BLK // _L, _ones, None, unroll=8)

    pltpu.sync_copy(z_v, acc_sh.at[pl.ds(s * _RPT, _RPT)])
    plsc.subcore_barrier()

    def _count(b, carry):
        pltpu.sync_copy(one_v, acc_sh.at[dst_v.at[b]], add=True)
        return carry
    lax.fori_loop(0, _NBLK, _count, None)

    plsc.subcore_barrier()
    pltpu.sync_copy(acc_sh.at[pl.ds(s * _RPT, _RPT)],
                    out_hbm.at[c, pl.ds(s * _RPT, _RPT)])


_GB = 16
_NBUF = 4
_RW = _D // 2
_NBLK_PAIR = 2 * _NBLK
_NBLK_C0 = 160
_NBLK_C1 = _NBLK_PAIR - _NBLK_C0


@functools.cache
def _sc_scatter_kernel():
    return functools.partial(
        pl.kernel,
        out_type=jax.ShapeDtypeStruct((_NC, _NP, _D), jnp.float32),
        mesh=_sc_mesh(),
        scratch_types=[
            pltpu.VMEM((2, _GB, _BLK), jnp.int32),
            pltpu.VMEM((2, _GB, _BLK), jnp.int32),
            pltpu.VMEM((_NBUF, _BLK, _RW), jnp.int32),
            pltpu.VMEM((3, _BLK, _D), jnp.float32),
            pltpu.VMEM_SHARED((_NP, _D), jnp.float32),
        ] + [pltpu.SemaphoreType.DMA] * (_NBUF + 4),
        compiler_params=pltpu.CompilerParams(use_tc_tiling_on_sc=False,
                                             needs_layout_passes=False),
    )(_sc_scatter_body)


def _sc_scatter_rows(gp, srcp, dstp):
    return _sc_scatter_kernel()(gp, srcp, dstp)


def _sc_scatter_body(g_hbm, src_hbm, dst_hbm, out_hbm,
                     src_v, dst_v, rows_v, frows_v, acc_sh, *sems):
    gsems, ssems, sem_i = sems[:_NBUF], sems[_NBUF:_NBUF + 3], sems[_NBUF + 3]
    c = lax.axis_index("c")
    s = lax.axis_index("s")
    blk0 = s * _NBLK_PAIR + lax.select(c == 0, 0, _NBLK_C0)
    ng = lax.select(c == 0, _NBLK_C0 // _GB, _NBLK_C1 // _GB)

    def _zero(t, carry):
        frows_v[0, t // (_D // _L), pl.ds((t % (_D // _L)) * _L, _L)] = (
            jnp.zeros((_L,), jnp.float32))
        return carry
    lax.fori_loop(0, _BLK * (_D // _L), _zero, None, unroll=8)
    for k in range(_RPT // _BLK):
        pltpu.sync_copy(frows_v.at[0],
                        acc_sh.at[pl.ds(s * _RPT + k * _BLK, _BLK)])
    plsc.subcore_barrier()

    pltpu.sync_copy(src_hbm.at[pl.ds(blk0, _GB)], src_v.at[0])
    pltpu.sync_copy(dst_hbm.at[pl.ds(blk0, _GB)], dst_v.at[0])

    def _group(gidx, carry):
        buf = gidx % 2
        nxt = 1 - buf

        @pl.when(gidx < ng - 1)
        def _():
            pltpu.async_copy(
                src_hbm.at[pl.ds(blk0 + (gidx + 1) * _GB, _GB)],
                src_v.at[nxt], sem_i)
            pltpu.async_copy(
                dst_hbm.at[pl.ds(blk0 + (gidx + 1) * _GB, _GB)],
                dst_v.at[nxt], sem_i)

        for k in range(_NBUF - 1):
            pltpu.async_copy(g_hbm.at[src_v.at[buf, k]], rows_v.at[k],
                             gsems[k])
        for b in range(_GB):
            k = b % _NBUF
            fb = b % 3
            if b + _NBUF - 1 < _GB:
                kr = (b + _NBUF - 1) % _NBUF
                pltpu.async_copy(g_hbm.at[src_v.at[buf, b + _NBUF - 1]],
                                 rows_v.at[kr], gsems[kr])
            pltpu.make_async_copy(g_hbm.at[src_v.at[buf, b]], rows_v.at[k],
                                  gsems[k]).wait()
            if b >= 3:
                pltpu.make_async_copy(frows_v.at[fb],
                                      acc_sh.at[dst_v.at[buf, b]],
                                      ssems[fb]).wait()

            def _conv(r, carry2, _k=k, _fb=fb):
                for j in range(_RW // _L):
                    v = rows_v[_k, r, pl.ds(j * _L, _L)]
                    frows_v[_fb, r, pl.ds(j * _L, _L)] = plsc.bitcast(
                        v << 16, jnp.float32)
                    frows_v[_fb, r, pl.ds(_RW + j * _L, _L)] = plsc.bitcast(
                        v & jnp.int32(-65536), jnp.float32)
                return carry2
            lax.fori_loop(0, _BLK, _conv, None, unroll=4)

            pltpu.async_copy(frows_v.at[fb], acc_sh.at[dst_v.at[buf, b]],
                             ssems[fb], add=True)
        for fb in range(3):
            pltpu.make_async_copy(frows_v.at[fb],
                                  acc_sh.at[dst_v.at[buf, 0]],
                                  ssems[fb]).wait()

        @pl.when(gidx < ng - 1)
        def _():
            pltpu.make_async_copy(src_hbm.at[pl.ds(0, _GB)],
                                  src_v.at[nxt], sem_i).wait()
            pltpu.make_async_copy(dst_hbm.at[pl.ds(0, _GB)],
                                  dst_v.at[nxt], sem_i).wait()
        return carry
    lax.fori_loop(0, ng, _group, None)

    plsc.subcore_barrier()
    pltpu.sync_copy(acc_sh.at[pl.ds(s * _RPT, _RPT)],
                    out_hbm.at[c, pl.ds(s * _RPT, _RPT)])


_R = 1024


def _pack_bf16(g):
    lo = lax.bitcast_convert_type(
        g[:, :_RW].astype(jnp.bfloat16), jnp.uint16).astype(jnp.uint32)
    hi = lax.bitcast_convert_type(
        g[:, _RW:].astype(jnp.bfloat16), jnp.uint16).astype(jnp.uint32)
    return lax.bitcast_convert_type(lo | (hi << 16), jnp.int32)


def _tc_first(cntT, xp, W1):
    def body(cnt_ref, x_ref, w_ref, dinv_ref, g_ref, gp_ref):
        deg = 1.0 + cnt_ref[:, 0:1] + cnt_ref[:, 1:2]
        dinv = lax.rsqrt(deg)
        dinv_ref[...] = dinv
        g = jnp.dot(x_ref[...], w_ref[...],
                    preferred_element_type=jnp.float32) * dinv
        g_ref[...] = g
        gp_ref[...] = _pack_bf16(g)
    return pl.pallas_call(
        body,
        grid=(_NP // _R,),
        in_specs=[pl.BlockSpec((_R, 2), lambda i: (i, 0)),
                  pl.BlockSpec((_R, _D), lambda i: (i, 0)),
                  pl.BlockSpec((_D, _D), lambda i: (0, 0))],
        out_specs=[pl.BlockSpec((_R, 1), lambda i: (i, 0)),
                   pl.BlockSpec((_R, _D), lambda i: (i, 0)),
                   pl.BlockSpec((_R, _RW), lambda i: (i, 0))],
        out_shape=[jax.ShapeDtypeStruct((_NP, 1), jnp.float32),
                   jax.ShapeDtypeStruct((_NP, _D), jnp.float32),
                   jax.ShapeDtypeStruct((_NP, _RW), jnp.int32)],
    )(cntT, xp, W1)


def _leaky(v):
    return jnp.where(v >= 0, v, 0.2 * v)


def _tc_combine(sp, g, dinv, b, hprev, Wn):
    has_res = hprev is not None

    def body(*refs):
        if has_res:
            (s_ref, g_ref, dinv_ref, b_ref, h_ref, w_ref,
             hout_ref, gout_ref, gpout_ref) = refs
        else:
            (s_ref, g_ref, dinv_ref, b_ref, w_ref,
             hout_ref, gout_ref, gpout_ref) = refs
        dinv = dinv_ref[...]
        conv = dinv * (s_ref[0] + s_ref[1] + g_ref[...]) + b_ref[...]
        pre = h_ref[...] + conv if has_res else conv
        h = _leaky(pre)
        hout_ref[...] = h
        gout = jnp.dot(h, w_ref[...],
                       preferred_element_type=jnp.float32) * dinv
        gout_ref[...] = gout
        gpout_ref[...] = _pack_bf16(gout)

    in_specs = [pl.BlockSpec((_NC, _R, _D), lambda i: (0, i, 0)),
                pl.BlockSpec((_R, _D), lambda i: (i, 0)),
                pl.BlockSpec((_R, 1), lambda i: (i, 0)),
                pl.BlockSpec((1, _D), lambda i: (0, 0))]
    args = [sp, g, dinv, b]
    if has_res:
        in_specs.append(pl.BlockSpec((_R, _D), lambda i: (i, 0)))
        args.append(hprev)
    in_specs.append(pl.BlockSpec((_D, _D), lambda i: (0, 0)))
    args.append(Wn)
    return pl.pallas_call(
        body,
        grid=(_NP // _R,),
        in_specs=in_specs,
        out_specs=[pl.BlockSpec((_R, _D), lambda i: (i, 0)),
                   pl.BlockSpec((_R, _D), lambda i: (i, 0)),
                   pl.BlockSpec((_R, _RW), lambda i: (i, 0))],
        out_shape=[jax.ShapeDtypeStruct((_NP, _D), jnp.float32),
                   jax.ShapeDtypeStruct((_NP, _D), jnp.float32),
                   jax.ShapeDtypeStruct((_NP, _RW), jnp.int32)],
    )(*args)


def _tc_final(sp, g, dinv, b, hprev, Wl, bl):
    def body(s_ref, g_ref, dinv_ref, b_ref, h_ref, wl_ref, bl_ref, out_ref):
        dinv = dinv_ref[...]
        conv = dinv * (s_ref[0] + s_ref[1] + g_ref[...]) + b_ref[...]
        h = _leaky(h_ref[...] + conv)
        out_ref[...] = jnp.dot(h, wl_ref[...],
                               preferred_element_type=jnp.float32) + bl_ref[...]
    return pl.pallas_call(
        body,
        grid=(_NP // _R,),
        in_specs=[pl.BlockSpec((_NC, _R, _D), lambda i: (0, i, 0)),
                  pl.BlockSpec((_R, _D), lambda i: (i, 0)),
                  pl.BlockSpec((_R, 1), lambda i: (i, 0)),
                  pl.BlockSpec((1, _D), lambda i: (0, 0)),
                  pl.BlockSpec((_R, _D), lambda i: (i, 0)),
                  pl.BlockSpec((_D, 1), lambda i: (0, 0)),
                  pl.BlockSpec((1, 1), lambda i: (0, 0))],
        out_specs=pl.BlockSpec((_R, 1), lambda i: (i, 0)),
        out_shape=jax.ShapeDtypeStruct((_NP, 1), jnp.float32),
    )(sp, g, dinv, b, hprev, Wl, bl)


def kernel(x, edge_index, W1, b1, W2, b2, W3, b3, Wl, bl):
    pad = jnp.full((_EP - _E,), _DUMMY, dtype=jnp.int32)
    src_flat = jnp.concatenate([edge_index[0], pad])
    dst_flat = jnp.concatenate([edge_index[1], pad])
    srcp = src_flat.reshape(_NS * _NBLK_PAIR, _BLK)
    dstp = dst_flat.reshape(_NS * _NBLK_PAIR, _BLK)
    xp = jnp.pad(x, ((0, _NP - _N), (0, 0)))

    cnt = _sc_degree(dst_flat.reshape(_NW, _NBLK, _BLK))
    dinv, g1, gp1 = _tc_first(cnt.T, xp, W1)
    s1 = _sc_scatter_rows(gp1, srcp, dstp)
    h1, g2, gp2 = _tc_combine(s1, g1, dinv, b1.reshape(1, _D), None, W2)
    s2 = _sc_scatter_rows(gp2, srcp, dstp)
    h2, g3, gp3 = _tc_combine(s2, g2, dinv, b2.reshape(1, _D), h1, W3)
    s3 = _sc_scatter_rows(gp3, srcp, dstp)
    logits = _tc_final(s3, g3, dinv, b3.reshape(1, _D), h2, Wl,
                       bl.reshape(1, 1))
    return logits[:_N, 0]
